# baseline (speedup 1.0000x reference)
import functools

import jax
import jax.numpy as jnp
import numpy as np
from jax import lax
from jax.experimental import pallas as pl
from jax.experimental.pallas import tpu as pltpu
from jax.experimental.pallas import tpu_sc as plsc

B, T, N, F, H, HEADS, HOR, E = 2, 12, 1024, 3, 128, 8, 12, 8192
DH = H // HEADS
NB = 256
SCALE = 1.0 / np.sqrt(DH)


_NC, _NS, _L = 2, 16, 16
_NW = _NC * _NS
_ROWS = N // _NW
_CHUNKS = E // _L


def _adj_body(src_hbm, dst_hbm, w_hbm, zeros_hbm, out_hbm, src_v, dst_v, w_v,
              acc_v):
    wid = lax.axis_index("s") * _NC + lax.axis_index("c")
    base = wid * _ROWS
    pltpu.sync_copy(src_hbm, src_v)
    pltpu.sync_copy(dst_hbm, dst_v)
    pltpu.sync_copy(w_hbm, w_v)
    pltpu.sync_copy(zeros_hbm, acc_v)

    def body(c, carry):
        off = c * _L
        dstc = dst_v[pl.ds(off, _L)]
        srcc = src_v[pl.ds(off, _L)]
        wc = w_v[pl.ds(off, _L)]
        msk = (dstc >= base) & (dstc < base + _ROWS)
        idx = (dstc - base) * N + srcc
        plsc.addupdate_scatter(acc_v, [idx], wc, mask=msk)
        return carry

    lax.fori_loop(0, _CHUNKS, body, 0)
    pltpu.sync_copy(acc_v, out_hbm.at[pl.ds(base * N, _ROWS * N)])


def _build_adj(src, dst, w):
    zeros = jnp.zeros((_ROWS * N,), jnp.float32)
    k = functools.partial(
        pl.kernel,
        mesh=plsc.VectorSubcoreMesh(core_axis_name="c", subcore_axis_name="s"),
        out_type=jax.ShapeDtypeStruct((N * N,), jnp.float32),
        compiler_params=pltpu.CompilerParams(needs_layout_passes=False),
        scratch_types=[
            pltpu.VMEM((E,), jnp.int32),
            pltpu.VMEM((E,), jnp.int32),
            pltpu.VMEM((E,), jnp.float32),
            pltpu.VMEM((_ROWS * N,), jnp.float32),
        ],
    )(_adj_body)
    return k(src, dst, w, zeros).reshape(N, N)


def _spatial_body(x_ref, cos_ref, sin_ref, adj_ref, we_ref, be_ref, wq_ref,
                  wk_ref, wv_ref, wo_ref, out_ref, lmask_ref):
    b = pl.program_id(0)
    t = pl.program_id(1)

    @pl.when((b == 0) & (t == 0))
    def _():
        lmask_ref[...] = jnp.log(adj_ref[...])

    xb = x_ref[0, 0]
    h = jnp.dot(xb, we_ref[...], preferred_element_type=jnp.float32)
    h = h + be_ref[...]
    cos = cos_ref[0]
    sin = sin_ref[0]
    hsw = jnp.concatenate([h[:, H // 2:], h[:, :H // 2]], axis=1)
    hr = h * cos + hsw * sin
    q = jnp.dot(hr, wq_ref[...], preferred_element_type=jnp.float32)
    k = jnp.dot(hr, wk_ref[...], preferred_element_type=jnp.float32)
    v = jnp.dot(hr, wv_ref[...], preferred_element_type=jnp.float32)
    q = q * SCALE
    lmask = lmask_ref[...]
    ones_col = jnp.ones((N, 1), jnp.float32)
    outs = []
    for i in range(HEADS):
        sl = slice(i * DH, (i + 1) * DH)
        qh, kh, vh = q[:, sl], k[:, sl], v[:, sl]
        s = lax.dot_general(qh, kh, (((1,), (1,)), ((), ())),
                            preferred_element_type=jnp.float32)
        c = s + lmask
        m = jnp.max(c, axis=1, keepdims=True)
        e = jnp.exp(c - jnp.maximum(m, -1e30))
        vext = jnp.concatenate([vh, ones_col], axis=1)
        nd = jnp.dot(e, vext, preferred_element_type=jnp.float32)
        outs.append(nd[:, :DH] / (nd[:, DH:DH + 1] + 1e-16))
    agg = jnp.concatenate(outs, axis=1)
    agg = jnp.dot(agg, wo_ref[...], preferred_element_type=jnp.float32)
    y = hr + agg
    mu = jnp.mean(y, axis=1, keepdims=True)
    var = jnp.mean((y - mu) ** 2, axis=1, keepdims=True)
    out_ref[0, 0] = (y - mu) * lax.rsqrt(var + 1e-5)


def _spatial_call(x, cos2, sin2, adj, We, be2, Wq, Wk, Wv, Wo):
    cw = lambda b, t: (b, t, 0, 0)
    ct = lambda b, t: (t, 0, 0)
    c0 = lambda b, t: (0, 0)
    return pl.pallas_call(
        _spatial_body,
        grid=(B, T),
        in_specs=[
            pl.BlockSpec((1, 1, N, F), cw),
            pl.BlockSpec((1, 1, H), ct),
            pl.BlockSpec((1, 1, H), ct),
            pl.BlockSpec((N, N), c0),
            pl.BlockSpec((F, H), c0),
            pl.BlockSpec((1, H), c0),
            pl.BlockSpec((H, H), c0),
            pl.BlockSpec((H, H), c0),
            pl.BlockSpec((H, H), c0),
            pl.BlockSpec((H, H), c0),
        ],
        out_specs=pl.BlockSpec((1, 1, N, H), cw),
        out_shape=jax.ShapeDtypeStruct((B, T, N, H), jnp.float32),
        scratch_shapes=[pltpu.VMEM((N, N), jnp.float32)],
    )(x, cos2, sin2, adj, We, be2, Wq, Wk, Wv, Wo)


def _temporal_body(h_ref, wtq_ref, wtk_ref, wtv_ref, wto_ref, wd_ref, bd_ref,
                   wh_ref, bh_ref, out_ref):
    hb = h_ref[0]
    h2 = hb.reshape(T * NB, H)
    tq = jnp.dot(h2, wtq_ref[...], preferred_element_type=jnp.float32)
    tk = jnp.dot(h2, wtk_ref[...], preferred_element_type=jnp.float32)
    tv = jnp.dot(h2, wtv_ref[...], preferred_element_type=jnp.float32)
    tq = tq * SCALE
    tq3 = tq.reshape(T, NB, H)
    tk3 = tk.reshape(T, NB, H)
    tv3 = tv.reshape(T, NB, H)
    d_idx = lax.broadcasted_iota(jnp.int32, (H, HEADS), 0)
    h_idx = lax.broadcasted_iota(jnp.int32, (H, HEADS), 1)
    hmat = (d_idx // DH == h_idx).astype(jnp.float32)
    TH = T * HEADS
    r_idx = lax.broadcasted_iota(jnp.int32, (HEADS, TH), 0)
    c_idx = lax.broadcasted_iota(jnp.int32, (HEADS, TH), 1)
    bca8 = (c_idx % HEADS == r_idx).astype(jnp.float32)
    red96 = jnp.transpose(bca8)
    g_in = lax.broadcasted_iota(jnp.int32, (TH, T * H), 0)
    g_out = lax.broadcasted_iota(jnp.int32, (TH, T * H), 1)
    gexp = ((g_in // HEADS == g_out // H) &
            ((g_out % H) // DH == g_in % HEADS)).astype(jnp.float32)
    touts = []
    for t in range(T):
        cs = [jnp.dot(tq3[t] * tk3[s], hmat,
                      preferred_element_type=jnp.float32) for s in range(T)]
        c96 = jnp.concatenate(cs, axis=1)
        m48 = jnp.maximum(c96[:, :48], c96[:, 48:])
        m24 = jnp.maximum(m48[:, :24], m48[:, 24:])
        m8 = jnp.maximum(jnp.maximum(m24[:, :8], m24[:, 8:16]), m24[:, 16:24])
        mb = jnp.dot(m8, bca8, preferred_element_type=jnp.float32)
        es96 = jnp.exp(c96 - mb)
        den8 = jnp.dot(es96, red96, preferred_element_type=jnp.float32)
        w96 = es96 * jnp.dot(1.0 / den8, bca8,
                             preferred_element_type=jnp.float32)
        wexp = jnp.dot(w96, gexp, preferred_element_type=jnp.float32)
        acc = wexp[:, :H] * tv3[0]
        for s in range(1, T):
            acc = acc + wexp[:, s * H:(s + 1) * H] * tv3[s]
        touts.append(acc)
    tout = jnp.concatenate([a[None] for a in touts], axis=0)
    to2 = jnp.dot(tout.reshape(T * NB, H), wto_ref[...],
                  preferred_element_type=jnp.float32)
    y = h2 + to2
    mu = jnp.mean(y, axis=1, keepdims=True)
    var = jnp.mean((y - mu) ** 2, axis=1, keepdims=True)
    hf = (y - mu) * lax.rsqrt(var + 1e-5)
    d = jnp.dot(hf, wd_ref[...], preferred_element_type=jnp.float32)
    d = d + bd_ref[...]
    d3 = d.reshape(T, NB, F)
    for hor in range(HOR):
        acc = d3[0] * wh_ref[0, hor]
        for t in range(1, T):
            acc = acc + d3[t] * wh_ref[t, hor]
        out_ref[0, hor] = acc + bh_ref[hor]


def _temporal_call(hsp, Wtq, Wtk, Wtv, Wto, Wd, bd2, Wh, bh):
    ch = lambda b, nb: (b, 0, nb, 0)
    c0 = lambda b, nb: (0, 0)
    return pl.pallas_call(
        _temporal_body,
        grid=(B, N // NB),
        in_specs=[
            pl.BlockSpec((1, T, NB, H), ch),
            pl.BlockSpec((H, H), c0),
            pl.BlockSpec((H, H), c0),
            pl.BlockSpec((H, H), c0),
            pl.BlockSpec((H, H), c0),
            pl.BlockSpec((H, F), c0),
            pl.BlockSpec((1, F), c0),
            pl.BlockSpec(memory_space=pltpu.SMEM),
            pl.BlockSpec(memory_space=pltpu.SMEM),
        ],
        out_specs=pl.BlockSpec((1, HOR, NB, F), ch),
        out_shape=jax.ShapeDtypeStruct((B, HOR, N, F), jnp.float32),
    )(hsp, Wtq, Wtk, Wtv, Wto, Wd, bd2, Wh, bh)


def _rope_tables():
    half = H // 2
    t = jnp.arange(T, dtype=jnp.float32)
    freqs = 1.0 / (10000.0 ** (jnp.arange(half, dtype=jnp.float32) / half))
    ang = t[:, None] * freqs[None, :]
    cos = jnp.cos(ang)
    sin = jnp.sin(ang)
    cos2 = jnp.concatenate([cos, cos], axis=1).reshape(T, 1, H)
    sin2 = jnp.concatenate([-sin, sin], axis=1).reshape(T, 1, H)
    return cos2, sin2


def kernel(x, edge_index, edge_weight, We, be, Wq, Wk, Wv, Wo, Wtq, Wtk, Wtv,
           Wto, Wd, bd, Wh, bh):
    adj = _build_adj(edge_index[0], edge_index[1], edge_weight)
    cos2, sin2 = _rope_tables()
    hsp = _spatial_call(x, cos2, sin2, adj, We, be.reshape(1, H), Wq, Wk, Wv,
                        Wo)
    return _temporal_call(hsp, Wtq, Wtk, Wtv, Wto, Wd, bd.reshape(1, F), Wh,
                          bh)


# speedup vs baseline: 1.1032x; 1.1032x over previous
import functools

import jax
import jax.numpy as jnp
import numpy as np
from jax import lax
from jax.experimental import pallas as pl
from jax.experimental.pallas import tpu as pltpu
from jax.experimental.pallas import tpu_sc as plsc

B, T, N, F, H, HEADS, HOR, E = 2, 12, 1024, 3, 128, 8, 12, 8192
DH = H // HEADS
NB = 256
SCALE = 1.0 / np.sqrt(DH)


_NC, _NS, _L = 2, 16, 16
_NW = _NC * _NS
_ROWS = N // _NW
_CHUNKS = E // _L


def _adj_body(src_hbm, dst_hbm, w_hbm, zeros_hbm, out_hbm, src_v, dst_v, w_v,
              acc_v):
    wid = lax.axis_index("s") * _NC + lax.axis_index("c")
    base = wid * _ROWS
    pltpu.sync_copy(src_hbm, src_v)
    pltpu.sync_copy(dst_hbm, dst_v)
    pltpu.sync_copy(w_hbm, w_v)
    pltpu.sync_copy(zeros_hbm, acc_v)

    def body(c, carry):
        off = c * _L
        dstc = dst_v[pl.ds(off, _L)]
        srcc = src_v[pl.ds(off, _L)]
        wc = w_v[pl.ds(off, _L)]
        msk = (dstc >= base) & (dstc < base + _ROWS)
        idx = (dstc - base) * N + srcc
        plsc.addupdate_scatter(acc_v, [idx], wc, mask=msk)
        return carry

    lax.fori_loop(0, _CHUNKS, body, 0)
    pltpu.sync_copy(acc_v, out_hbm.at[pl.ds(base * N, _ROWS * N)])


def _build_adj(src, dst, w):
    zeros = jnp.zeros((_ROWS * N,), jnp.float32)
    k = functools.partial(
        pl.kernel,
        mesh=plsc.VectorSubcoreMesh(core_axis_name="c", subcore_axis_name="s"),
        out_type=jax.ShapeDtypeStruct((N * N,), jnp.float32),
        compiler_params=pltpu.CompilerParams(needs_layout_passes=False),
        scratch_types=[
            pltpu.VMEM((E,), jnp.int32),
            pltpu.VMEM((E,), jnp.int32),
            pltpu.VMEM((E,), jnp.float32),
            pltpu.VMEM((_ROWS * N,), jnp.float32),
        ],
    )(_adj_body)
    return k(src, dst, w, zeros).reshape(N, N)


def _spatial_body(x_ref, cos_ref, sin_ref, adj_ref, we_ref, be_ref, wq_ref,
                  wk_ref, wv_ref, wo_ref, out_ref, lmask_ref):
    b = pl.program_id(0)
    t = pl.program_id(1)

    @pl.when((b == 0) & (t == 0))
    def _():
        lmask_ref[...] = jnp.log(adj_ref[...])

    xb = x_ref[0, 0]
    h = jnp.dot(xb, we_ref[...], preferred_element_type=jnp.float32)
    h = h + be_ref[...]
    cos = cos_ref[0]
    sin = sin_ref[0]
    hsw = jnp.concatenate([h[:, H // 2:], h[:, :H // 2]], axis=1)
    hr = h * cos + hsw * sin
    q = jnp.dot(hr, wq_ref[...], preferred_element_type=jnp.float32)
    k = jnp.dot(hr, wk_ref[...], preferred_element_type=jnp.float32)
    v = jnp.dot(hr, wv_ref[...], preferred_element_type=jnp.float32)
    lmask = lmask_ref[...]
    ones_col = jnp.ones((N, 1), jnp.float32)
    outs = []
    for i in range(HEADS):
        sl = slice(i * DH, (i + 1) * DH)
        qh, kh, vh = q[:, sl], k[:, sl], v[:, sl]
        s = lax.dot_general(qh, kh, (((1,), (1,)), ((), ())),
                            preferred_element_type=jnp.float32)
        c = s + lmask
        m = jnp.max(c, axis=1, keepdims=True)
        e = jnp.exp(c - jnp.maximum(m, -1e30))
        vext = jnp.concatenate([vh, ones_col], axis=1)
        nd = jnp.dot(e, vext, preferred_element_type=jnp.float32)
        outs.append(nd[:, :DH] / (nd[:, DH:DH + 1] + 1e-16))
    agg = jnp.concatenate(outs, axis=1)
    agg = jnp.dot(agg, wo_ref[...], preferred_element_type=jnp.float32)
    y = hr + agg
    mu = jnp.mean(y, axis=1, keepdims=True)
    var = jnp.mean((y - mu) ** 2, axis=1, keepdims=True)
    out_ref[0, 0] = (y - mu) * lax.rsqrt(var + 1e-5)


def _spatial_call(x, cos2, sin2, adj, We, be2, Wq, Wk, Wv, Wo):
    cw = lambda b, t: (b, t, 0, 0)
    ct = lambda b, t: (t, 0, 0)
    c0 = lambda b, t: (0, 0)
    return pl.pallas_call(
        _spatial_body,
        grid=(B, T),
        in_specs=[
            pl.BlockSpec((1, 1, N, F), cw),
            pl.BlockSpec((1, 1, H), ct),
            pl.BlockSpec((1, 1, H), ct),
            pl.BlockSpec((N, N), c0),
            pl.BlockSpec((F, H), c0),
            pl.BlockSpec((1, H), c0),
            pl.BlockSpec((H, H), c0),
            pl.BlockSpec((H, H), c0),
            pl.BlockSpec((H, H), c0),
            pl.BlockSpec((H, H), c0),
        ],
        out_specs=pl.BlockSpec((1, 1, N, H), cw),
        out_shape=jax.ShapeDtypeStruct((B, T, N, H), jnp.float32),
        scratch_shapes=[pltpu.VMEM((N, N), jnp.float32)],
    )(x, cos2, sin2, adj, We, be2, Wq, Wk, Wv, Wo)


def _temporal_body(h_ref, wtq_ref, wtk_ref, wtv_ref, wto_ref, wd_ref, bd_ref,
                   wh_ref, bh_ref, out_ref):
    hb = h_ref[0]
    h2 = hb.reshape(T * NB, H)
    tq = jnp.dot(h2, wtq_ref[...], preferred_element_type=jnp.float32)
    tk = jnp.dot(h2, wtk_ref[...], preferred_element_type=jnp.float32)
    tv = jnp.dot(h2, wtv_ref[...], preferred_element_type=jnp.float32)
    tq3 = tq.reshape(T, NB, H)
    tk3 = tk.reshape(T, NB, H)
    tv3 = tv.reshape(T, NB, H)
    d_idx = lax.broadcasted_iota(jnp.int32, (H, HEADS), 0)
    h_idx = lax.broadcasted_iota(jnp.int32, (H, HEADS), 1)
    hmat = (d_idx // DH == h_idx).astype(jnp.float32)
    hexp = jnp.transpose(hmat)
    rows = [[None] * T for _ in range(T)]
    for t in range(T):
        for s in range(T):
            rows[t][s] = jnp.dot(tq3[t] * tk3[s], hmat,
                                 preferred_element_type=jnp.float32)
    touts = []
    for t in range(T):
        m = rows[t][0]
        for s in range(1, T):
            m = jnp.maximum(m, rows[t][s])
        es = [jnp.exp(rows[t][s] - m) for s in range(T)]
        den = es[0]
        for s in range(1, T):
            den = den + es[s]
        recip = 1.0 / den
        acc = None
        for s in range(T):
            a = jnp.dot(es[s] * recip, hexp,
                        preferred_element_type=jnp.float32)
            term = a * tv3[s]
            acc = term if acc is None else acc + term
        touts.append(acc)
    tout = jnp.concatenate([a[None] for a in touts], axis=0)
    to2 = jnp.dot(tout.reshape(T * NB, H), wto_ref[...],
                  preferred_element_type=jnp.float32)
    y = h2 + to2
    mu = jnp.mean(y, axis=1, keepdims=True)
    var = jnp.mean((y - mu) ** 2, axis=1, keepdims=True)
    hf = (y - mu) * lax.rsqrt(var + 1e-5)
    d = jnp.dot(hf, wd_ref[...], preferred_element_type=jnp.float32)
    d = d + bd_ref[...]
    d3 = d.reshape(T, NB, F)
    for hor in range(HOR):
        acc = d3[0] * wh_ref[0, hor]
        for t in range(1, T):
            acc = acc + d3[t] * wh_ref[t, hor]
        out_ref[0, hor] = acc + bh_ref[hor]


def _temporal_call(hsp, Wtq, Wtk, Wtv, Wto, Wd, bd2, Wh, bh):
    ch = lambda b, nb: (b, 0, nb, 0)
    c0 = lambda b, nb: (0, 0)
    return pl.pallas_call(
        _temporal_body,
        grid=(B, N // NB),
        in_specs=[
            pl.BlockSpec((1, T, NB, H), ch),
            pl.BlockSpec((H, H), c0),
            pl.BlockSpec((H, H), c0),
            pl.BlockSpec((H, H), c0),
            pl.BlockSpec((H, H), c0),
            pl.BlockSpec((H, F), c0),
            pl.BlockSpec((1, F), c0),
            pl.BlockSpec(memory_space=pltpu.SMEM),
            pl.BlockSpec(memory_space=pltpu.SMEM),
        ],
        out_specs=pl.BlockSpec((1, HOR, NB, F), ch),
        out_shape=jax.ShapeDtypeStruct((B, HOR, N, F), jnp.float32),
    )(hsp, Wtq, Wtk, Wtv, Wto, Wd, bd2, Wh, bh)


def _rope_tables():
    half = H // 2
    t = jnp.arange(T, dtype=jnp.float32)
    freqs = 1.0 / (10000.0 ** (jnp.arange(half, dtype=jnp.float32) / half))
    ang = t[:, None] * freqs[None, :]
    cos = jnp.cos(ang)
    sin = jnp.sin(ang)
    cos2 = jnp.concatenate([cos, cos], axis=1).reshape(T, 1, H)
    sin2 = jnp.concatenate([-sin, sin], axis=1).reshape(T, 1, H)
    return cos2, sin2


def kernel(x, edge_index, edge_weight, We, be, Wq, Wk, Wv, Wo, Wtq, Wtk, Wtv,
           Wto, Wd, bd, Wh, bh):
    adj = _build_adj(edge_index[0], edge_index[1], edge_weight)
    cos2, sin2 = _rope_tables()
    Wq = Wq * SCALE
    Wtq = Wtq * SCALE
    hsp = _spatial_call(x, cos2, sin2, adj, We, be.reshape(1, H), Wq, Wk, Wv,
                        Wo)
    return _temporal_call(hsp, Wtq, Wtk, Wtv, Wto, Wd, bd.reshape(1, F), Wh,
                          bh)
